# Initial kernel scaffold; baseline (speedup 1.0000x reference)
import jax
import jax.numpy as jnp
from jax.experimental import pallas as pl


def kernel(x, W, b):
    raise NotImplementedError("write your pallas kernel here")


# speedup vs baseline: 1.3447x; 1.3447x over previous
import functools

import jax
import jax.numpy as jnp
from jax import lax
from jax.experimental import pallas as pl
from jax.experimental.pallas import tpu as pltpu
from jax.experimental.pallas import tpu_sc as plsc

B, S, D = 4, 4096, 4096
K = 2867
R = B * K
NW = 32
RPW = 360
RP = NW * RPW
SB = 512
SBS = 256
CH = 8


def _score_kernel(x_ref, w_ref, b_ref, out_ref):
    cols = [jnp.dot(x_ref[i], w_ref[...], preferred_element_type=jnp.float32)
            for i in range(B)]
    out_ref[...] = jnp.concatenate(cols, axis=1) + b_ref[0, 0]


def _rank_idx_kernel(srow_ref, scol_ref, idxt_ref):
    i_row = lax.broadcasted_iota(jnp.int32, (1, S), 1)
    for bb in range(B):
        sr = srow_ref[pl.ds(bb, 1), :]
        acc = jnp.zeros((1, S), jnp.int32)
        for c in range(S // SB):
            sc = scol_ref[pl.ds(c * SB, SB), pl.ds(bb, 1)]
            jio = lax.broadcasted_iota(jnp.int32, (SB, 1), 0) + c * SB
            beats = (sc > sr) | ((sc == sr) & (jio < i_row))
            acc = acc + jnp.sum(beats.astype(jnp.int32), axis=0, keepdims=True)
        for c in range(S // SB):
            pio = lax.broadcasted_iota(jnp.int32, (SB, 1), 0) + c * SB
            m = (acc == pio)
            contrib = jnp.sum(m.astype(jnp.int32) * i_row, axis=1, keepdims=True)
            idxt_ref[pl.ds(c * SB, SB), pl.ds(bb, 1)] = contrib


def _gather_kernel(x_hbm, gidx_hbm, out_hbm, idx_v, buf, sem):
    wid = lax.axis_index("s") * 2 + lax.axis_index("c")
    base = wid * RPW
    nw = jnp.minimum(RPW, R - base)
    pltpu.sync_copy(gidx_hbm.at[pl.ds(base, RPW)], idx_v)
    nfull = nw // CH

    def body(j, carry):
        pltpu.async_copy(x_hbm.at[idx_v.at[pl.ds(j * CH, CH)]], buf, sem).wait()
        pltpu.sync_copy(buf, out_hbm.at[pl.ds(base + j * CH, CH)])
        return carry

    lax.fori_loop(0, nfull, body, 0)

    tail = nw - nfull * CH
    @pl.when(tail > 0)
    def _():
        off = nfull * CH
        pltpu.async_copy(x_hbm.at[idx_v.at[pl.ds(off, CH)]], buf, sem).wait()
        pltpu.sync_copy(buf.at[pl.ds(0, 4)], out_hbm.at[pl.ds(base + off, 4)])


def _scores_call(x, W, b):
    return pl.pallas_call(
        _score_kernel,
        grid=(S // SBS,),
        in_specs=[
            pl.BlockSpec((B, SBS, D), lambda s: (0, s, 0)),
            pl.BlockSpec((D, 1), lambda s: (0, 0)),
            pl.BlockSpec((1, 1), lambda s: (0, 0)),
        ],
        out_specs=pl.BlockSpec((SBS, B), lambda s: (s, 0)),
        out_shape=jax.ShapeDtypeStruct((S, B), jnp.float32),
    )(x, W, b.reshape(1, 1))


def _rank_idx_call(srow, scol):
    return pl.pallas_call(
        _rank_idx_kernel,
        in_specs=[
            pl.BlockSpec((B, S), lambda: (0, 0)),
            pl.BlockSpec((S, B), lambda: (0, 0)),
        ],
        out_specs=pl.BlockSpec((S, B), lambda: (0, 0)),
        out_shape=jax.ShapeDtypeStruct((S, B), jnp.int32),
    )(srow, scol)


@functools.lru_cache(maxsize=1)
def _gather_call():
    return pl.kernel(
        _gather_kernel,
        out_type=jax.ShapeDtypeStruct((R, D), jnp.float32),
        mesh=plsc.VectorSubcoreMesh(core_axis_name="c", subcore_axis_name="s"),
        scratch_types=[
            pltpu.VMEM((RPW,), jnp.int32),
            pltpu.VMEM((CH, D), jnp.float32),
            pltpu.SemaphoreType.DMA,
        ],
    )


def kernel(x, W, b):
    scol = _scores_call(x, W, b)
    srow = scol.T
    idxt = _rank_idx_call(srow, scol)
    idx = idxt.T[:, :K]
    gidx = idx + (jnp.arange(B, dtype=jnp.int32) * S)[:, None]
    gidx = jnp.concatenate(
        [gidx.reshape(-1), jnp.zeros((RP - R,), jnp.int32)])
    out_flat = _gather_call()(x.reshape(B * S, D), gidx)
    return out_flat.reshape(B, K, D)


# speedup vs baseline: 1.4111x; 1.0494x over previous
import functools

import jax
import jax.numpy as jnp
from jax import lax
from jax.experimental import pallas as pl
from jax.experimental.pallas import tpu as pltpu
from jax.experimental.pallas import tpu_sc as plsc

B, S, D = 4, 4096, 4096
K = 2867
R = B * K
NW = 32
RPW = 360
RP = NW * RPW
SB = 512
SBS = 256
CH = 8


def _score_kernel(x_ref, w_ref, b_ref, out_ref):
    cols = [jnp.dot(x_ref[i], w_ref[...], preferred_element_type=jnp.float32)
            for i in range(B)]
    out_ref[...] = jnp.concatenate(cols, axis=1) + b_ref[0, 0]


def _rank_idx_kernel(srow_ref, scol_ref, idxt_ref):
    i_row = lax.broadcasted_iota(jnp.int32, (1, S), 1)
    for bb in range(B):
        sr = srow_ref[pl.ds(bb, 1), :]
        acc = jnp.zeros((1, S), jnp.int32)
        for c in range(S // SB):
            sc = scol_ref[pl.ds(c * SB, SB), pl.ds(bb, 1)]
            jio = lax.broadcasted_iota(jnp.int32, (SB, 1), 0) + c * SB
            beats = (sc > sr) | ((sc == sr) & (jio < i_row))
            acc = acc + jnp.sum(beats.astype(jnp.int32), axis=0, keepdims=True)
        for c in range(S // SB):
            pio = lax.broadcasted_iota(jnp.int32, (SB, 1), 0) + c * SB
            m = (acc == pio)
            contrib = jnp.sum(m.astype(jnp.int32) * i_row, axis=1, keepdims=True)
            idxt_ref[pl.ds(c * SB, SB), pl.ds(bb, 1)] = contrib


def _gather_kernel(x_hbm, gidx_hbm, out_hbm, idx_v,
                   buf0, buf1, gs0, gs1, os0, os1):
    wid = lax.axis_index("s") * 2 + lax.axis_index("c")
    base = wid * RPW
    nw = jnp.minimum(RPW, R - base)
    pltpu.sync_copy(gidx_hbm.at[pl.ds(base, RPW)], idx_v)
    nfull = nw // CH

    def gather(j, buf, sem):
        pltpu.async_copy(x_hbm.at[idx_v.at[pl.ds(j * CH, CH)]], buf, sem)

    def put(j, buf, sem):
        pltpu.async_copy(buf, out_hbm.at[pl.ds(base + j * CH, CH)], sem)

    def wait_gather(buf, sem):
        pltpu.make_async_copy(x_hbm.at[pl.ds(0, CH)], buf, sem).wait()

    def wait_put(buf, sem):
        pltpu.make_async_copy(buf, out_hbm.at[pl.ds(base, CH)], sem).wait()

    gather(0, buf0, gs0)
    gather(1, buf1, gs1)

    def body(g, carry):
        j0 = 2 * g
        j1 = j0 + 1

        wait_gather(buf0, gs0)
        put(j0, buf0, os0)

        @pl.when(j0 + 2 < nfull)
        def _():
            wait_put(buf0, os0)
            gather(j0 + 2, buf0, gs0)

        @pl.when(j1 < nfull)
        def _():
            wait_gather(buf1, gs1)
            put(j1, buf1, os1)

            @pl.when(j1 + 2 < nfull)
            def _():
                wait_put(buf1, os1)
                gather(j1 + 2, buf1, gs1)

        return carry

    lax.fori_loop(0, (nfull + 1) // 2, body, 0)
    wait_put(buf0, os0)
    wait_put(buf1, os1)

    tail = nw - nfull * CH
    @pl.when(tail > 0)
    def _():
        off = nfull * CH
        pltpu.async_copy(x_hbm.at[idx_v.at[pl.ds(off, CH)]], buf0, gs0).wait()
        pltpu.sync_copy(buf0.at[pl.ds(0, 4)], out_hbm.at[pl.ds(base + off, 4)])


def _scores_call(x, W, b):
    return pl.pallas_call(
        _score_kernel,
        grid=(S // SBS,),
        in_specs=[
            pl.BlockSpec((B, SBS, D), lambda s: (0, s, 0)),
            pl.BlockSpec((D, 1), lambda s: (0, 0)),
            pl.BlockSpec((1, 1), lambda s: (0, 0)),
        ],
        out_specs=pl.BlockSpec((SBS, B), lambda s: (s, 0)),
        out_shape=jax.ShapeDtypeStruct((S, B), jnp.float32),
    )(x, W, b.reshape(1, 1))


def _rank_idx_call(srow, scol):
    return pl.pallas_call(
        _rank_idx_kernel,
        in_specs=[
            pl.BlockSpec((B, S), lambda: (0, 0)),
            pl.BlockSpec((S, B), lambda: (0, 0)),
        ],
        out_specs=pl.BlockSpec((S, B), lambda: (0, 0)),
        out_shape=jax.ShapeDtypeStruct((S, B), jnp.int32),
    )(srow, scol)


@functools.lru_cache(maxsize=1)
def _gather_call():
    return pl.kernel(
        _gather_kernel,
        out_type=jax.ShapeDtypeStruct((R, D), jnp.float32),
        mesh=plsc.VectorSubcoreMesh(core_axis_name="c", subcore_axis_name="s"),
        scratch_types=[
            pltpu.VMEM((RPW,), jnp.int32),
            pltpu.VMEM((CH, D), jnp.float32),
            pltpu.VMEM((CH, D), jnp.float32),
            pltpu.SemaphoreType.DMA,
            pltpu.SemaphoreType.DMA,
            pltpu.SemaphoreType.DMA,
            pltpu.SemaphoreType.DMA,
        ],
    )


def kernel(x, W, b):
    scol = _scores_call(x, W, b)
    srow = scol.T
    idxt = _rank_idx_call(srow, scol)
    idx = idxt.T[:, :K]
    gidx = idx + (jnp.arange(B, dtype=jnp.int32) * S)[:, None]
    gidx = jnp.concatenate(
        [gidx.reshape(-1), jnp.zeros((RP - R,), jnp.int32)])
    out_flat = _gather_call()(x.reshape(B * S, D), gidx)
    return out_flat.reshape(B, K, D)


# speedup vs baseline: 1.6098x; 1.1408x over previous
import functools

import jax
import jax.numpy as jnp
from jax import lax
from jax.experimental import pallas as pl
from jax.experimental.pallas import tpu as pltpu
from jax.experimental.pallas import tpu_sc as plsc

B, S, D = 4, 4096, 4096
K = 2867
R = B * K
NW = 32
WPB = 8
RPW = 360
KP = WPB * RPW
SB = 512
SBS = 256
CH = 8


def _score_kernel(x_ref, w_ref, b_ref, out_ref):
    cols = [jnp.dot(x_ref[i], w_ref[...], preferred_element_type=jnp.float32)
            for i in range(B)]
    out_ref[...] = jnp.concatenate(cols, axis=1) + b_ref[0, 0]


def _rank_idx_kernel(srow_ref, scol_ref, idxt_ref):
    i_row = lax.broadcasted_iota(jnp.int32, (1, S), 1)
    for bb in range(B):
        sr = srow_ref[pl.ds(bb, 1), :]
        acc = jnp.zeros((1, S), jnp.int32)
        for c in range(S // SB):
            sc = scol_ref[pl.ds(c * SB, SB), pl.ds(bb, 1)]
            jio = lax.broadcasted_iota(jnp.int32, (SB, 1), 0) + c * SB
            beats = (sc > sr) | ((sc == sr) & (jio < i_row))
            acc = acc + jnp.sum(beats.astype(jnp.int32), axis=0, keepdims=True)
        for c in range(S // SB):
            pio = lax.broadcasted_iota(jnp.int32, (SB, 1), 0) + c * SB
            m = (acc == pio)
            contrib = jnp.sum(m.astype(jnp.int32) * i_row, axis=1, keepdims=True)
            idxt_ref[pl.ds(c * SB, SB), pl.ds(bb, 1)] = contrib


def _gather_kernel(x_hbm, gidx_hbm, out_hbm, idx_v,
                   buf0, buf1, gs0, gs1, os0, os1):
    wid = lax.axis_index("s") * 2 + lax.axis_index("c")
    bb = wid // WPB
    sub = wid % WPB
    base = sub * RPW
    out2 = out_hbm.at[bb]
    nw = jnp.minimum(RPW, K - base)
    pltpu.sync_copy(gidx_hbm.at[pl.ds(bb * KP + base, RPW)], idx_v)
    nfull = nw // CH

    def gather(j, buf, sem):
        pltpu.async_copy(x_hbm.at[idx_v.at[pl.ds(j * CH, CH)]], buf, sem)

    def put(j, buf, sem):
        pltpu.async_copy(buf, out2.at[pl.ds(base + j * CH, CH)], sem)

    def wait_gather(buf, sem):
        pltpu.make_async_copy(x_hbm.at[pl.ds(0, CH)], buf, sem).wait()

    def wait_put(buf, sem):
        pltpu.make_async_copy(buf, out2.at[pl.ds(base, CH)], sem).wait()

    gather(0, buf0, gs0)
    gather(1, buf1, gs1)

    def body(g, carry):
        j0 = 2 * g
        j1 = j0 + 1

        wait_gather(buf0, gs0)
        put(j0, buf0, os0)

        @pl.when(j0 + 2 < nfull)
        def _():
            wait_put(buf0, os0)
            gather(j0 + 2, buf0, gs0)

        @pl.when(j1 < nfull)
        def _():
            wait_gather(buf1, gs1)
            put(j1, buf1, os1)

            @pl.when(j1 + 2 < nfull)
            def _():
                wait_put(buf1, os1)
                gather(j1 + 2, buf1, gs1)

        return carry

    lax.fori_loop(0, (nfull + 1) // 2, body, 0)
    wait_put(buf0, os0)
    wait_put(buf1, os1)

    @pl.when(sub == WPB - 1)
    def _():
        off = (K - 3) - base
        pltpu.async_copy(x_hbm.at[idx_v.at[pl.ds(off, CH)]], buf0, gs0).wait()
        pltpu.sync_copy(buf0.at[pl.ds(0, 3)], out2.at[pl.ds(K - 3, 3)])


def _scores_call(x, W, b):
    return pl.pallas_call(
        _score_kernel,
        grid=(S // SBS,),
        in_specs=[
            pl.BlockSpec((B, SBS, D), lambda s: (0, s, 0)),
            pl.BlockSpec((D, 1), lambda s: (0, 0)),
            pl.BlockSpec((1, 1), lambda s: (0, 0)),
        ],
        out_specs=pl.BlockSpec((SBS, B), lambda s: (s, 0)),
        out_shape=jax.ShapeDtypeStruct((S, B), jnp.float32),
    )(x, W, b.reshape(1, 1))


def _rank_idx_call(srow, scol):
    return pl.pallas_call(
        _rank_idx_kernel,
        in_specs=[
            pl.BlockSpec((B, S), lambda: (0, 0)),
            pl.BlockSpec((S, B), lambda: (0, 0)),
        ],
        out_specs=pl.BlockSpec((S, B), lambda: (0, 0)),
        out_shape=jax.ShapeDtypeStruct((S, B), jnp.int32),
    )(srow, scol)


@functools.lru_cache(maxsize=1)
def _gather_call():
    return pl.kernel(
        _gather_kernel,
        out_type=jax.ShapeDtypeStruct((B, K, D), jnp.float32),
        mesh=plsc.VectorSubcoreMesh(core_axis_name="c", subcore_axis_name="s"),
        scratch_types=[
            pltpu.VMEM((RPW,), jnp.int32),
            pltpu.VMEM((CH, D), jnp.float32),
            pltpu.VMEM((CH, D), jnp.float32),
            pltpu.SemaphoreType.DMA,
            pltpu.SemaphoreType.DMA,
            pltpu.SemaphoreType.DMA,
            pltpu.SemaphoreType.DMA,
        ],
    )


def kernel(x, W, b):
    scol = _scores_call(x, W, b)
    srow = scol.T
    idxt = _rank_idx_call(srow, scol)
    idx = idxt.T[:, :K]
    gidx = idx + (jnp.arange(B, dtype=jnp.int32) * S)[:, None]
    gidx = jnp.pad(gidx, ((0, 0), (0, KP - K))).reshape(-1)
    return _gather_call()(x.reshape(B * S, D), gidx)
